# Initial kernel scaffold; baseline (speedup 1.0000x reference)
import jax
import jax.numpy as jnp
from jax.experimental import pallas as pl


def kernel(features, hand_tokens, action_mask, hand_ids, card_embedding_weight, pick_W, pick_b, partner_W, partner_b, pu_W, pu_b, ptr_Wg_W, ptr_Wg_b, ptr_Wt_W, ptr_Wt_b, ptr_v_W, tw_Wg_W, tw_Wg_b, tw_We_W, tw_We_b):
    raise NotImplementedError("write your pallas kernel here")


# speedup vs baseline: 10.5805x; 10.5805x over previous
import functools

import jax
import jax.numpy as jnp
from jax import lax
from jax.experimental import pallas as pl

A = 111
NEG = -100000000.0
PADNEG = -2.0e8
AP = 128


def _fused_kernel(feat_ref, tok_ref, mask_ref, hid_ref,
                  wcat_ref, bcat_ref, card_ref, twe_w_ref, twe_b_ref,
                  wt_ref, bt_ref, v_ref, out_ref):
    f = feat_ref[...]
    bm = f.shape[0]

    r = jnp.dot(f, wcat_ref[...], preferred_element_type=jnp.float32) + bcat_ref[...]
    g = r[:, 0:64]
    q = r[:, 64:128]

    kc = jnp.dot(card_ref[...], twe_w_ref[...],
                 preferred_element_type=jnp.float32) + twe_b_ref[...]
    calls = lax.dot_general(q, kc, (((1,), (1,)), ((), ())),
                            preferred_element_type=jnp.float32)

    wt = wt_ref[...]
    bt = bt_ref[...]
    v = v_ref[...]
    s_slots = []
    for i in range(8):
        ti = tok_ref[:, 64 * i:64 * (i + 1)]
        h = jnp.tanh(g + jnp.dot(ti, wt, preferred_element_type=jnp.float32) + bt)
        s_slots.append(jnp.dot(h, v, preferred_element_type=jnp.float32))

    col = lax.broadcasted_iota(jnp.int32, (bm, AP), 1)
    cid_col = jnp.where(col >= 76, col - 76, jnp.where(col >= 42, col - 42, col - 8))
    band = (col >= 8) & (col <= 109)

    logits = jnp.full((bm, AP), NEG, dtype=jnp.float32)
    logits = jnp.where(col == 0, r[:, 128:129], logits)
    logits = jnp.where(col == 1, r[:, 129:130], logits)
    logits = jnp.where(col == 2, r[:, 130:131], logits)
    logits = jnp.where(col == 3, r[:, 131:132], logits)
    logits = jnp.where(col == 110, r[:, 132:133], logits)
    for c in range(4):
        logits = jnp.where(col == 4 + c, calls[:, c:c + 1], logits)
    for i in range(8):
        ids_i = hid_ref[:, i:i + 1]
        logits = jnp.where(band & (cid_col == ids_i), s_slots[i], logits)

    logits = jnp.where(mask_ref[...], logits, NEG)
    logits = jnp.where(col < A, logits, PADNEG)

    m = jnp.max(logits, axis=1, keepdims=True)
    e = jnp.exp(logits - m)
    out_ref[...] = e / jnp.sum(e, axis=1, keepdims=True)


@functools.partial(jax.jit, static_argnames=())
def kernel(features, hand_tokens, action_mask, hand_ids, card_embedding_weight,
           pick_W, pick_b, partner_W, partner_b, pu_W, pu_b,
           ptr_Wg_W, ptr_Wg_b, ptr_Wt_W, ptr_Wt_b, ptr_v_W,
           tw_Wg_W, tw_Wg_b, tw_We_W, tw_We_b):
    B = features.shape[0]
    BM = 512
    grid = (B // BM,)

    tokens2 = hand_tokens.reshape(B, 8 * 64)
    maskp = jnp.pad(action_mask, ((0, 0), (0, AP - A)))
    hid = hand_ids.astype(jnp.int32)

    wcat = jnp.concatenate([ptr_Wg_W, tw_Wg_W, pick_W, partner_W, pu_W], axis=1)
    wcat = jnp.pad(wcat, ((0, 0), (0, 136 - 133)))
    bcat = jnp.concatenate([ptr_Wg_b, tw_Wg_b, pick_b, partner_b, pu_b])
    bcat = jnp.pad(bcat, (0, 136 - 133)).reshape(1, 136)

    card_sub = jnp.pad(card_embedding_weight[10:14], ((0, 4), (0, 0)))

    row_spec = lambda w: pl.BlockSpec((BM, w), lambda i: (i, 0))
    full = lambda a: pl.BlockSpec(a.shape, lambda i: (0,) * a.ndim)

    twe_b = tw_We_b.reshape(1, 64)
    bt = ptr_Wt_b.reshape(1, 64)

    out = pl.pallas_call(
        _fused_kernel,
        grid=grid,
        in_specs=[
            row_spec(256),
            row_spec(512),
            row_spec(AP),
            row_spec(8),
            full(wcat), full(bcat), full(card_sub), full(tw_We_W), full(twe_b),
            full(ptr_Wt_W), full(bt), full(ptr_v_W),
        ],
        out_specs=pl.BlockSpec((BM, AP), lambda i: (i, 0)),
        out_shape=jax.ShapeDtypeStruct((B, AP), jnp.float32),
    )(features, tokens2, maskp, hid,
      wcat, bcat, card_sub, tw_We_W, twe_b, ptr_Wt_W, bt, ptr_v_W)
    return out[:, :A]
